# baseline (speedup 1.0000x reference)
import functools

import jax
import jax.numpy as jnp
from jax import lax
from jax.experimental import pallas as pl
from jax.experimental.pallas import tpu as pltpu
from jax.experimental.pallas import tpu_sc as plsc

T_ST, H_ST, W_ST = 16, 64, 64
HIDDEN = 768
D3 = HIDDEN // 3
BH = 16
NB = H_ST // BH
NBUF = 4
GRID = T_ST * NB
H_OFF = 16
W_OFF = 128
EMB_ROWS = W_OFF + W_ST


def _sc_gather(t_idx, h_idx, w_idx, temporal, height, width):
    mesh = plsc.VectorSubcoreMesh(core_axis_name="c", subcore_axis_name="s")

    @functools.partial(
        pl.kernel,
        mesh=mesh,
        out_type=jax.ShapeDtypeStruct((EMB_ROWS, D3), jnp.float32),
        scratch_types=(
            pltpu.VMEM((T_ST,), jnp.int32),
            pltpu.VMEM((H_ST,), jnp.int32),
            pltpu.VMEM((W_ST,), jnp.int32),
            pltpu.VMEM((T_ST, D3), jnp.float32),
            pltpu.VMEM((H_ST, D3), jnp.float32),
            pltpu.VMEM((W_ST, D3), jnp.float32),
            pltpu.SemaphoreType.DMA,
        ),
    )
    def k(t_idx_hbm, h_idx_hbm, w_idx_hbm, t_hbm, h_hbm, w_hbm, out,
          t_idx_v, h_idx_v, w_idx_v, t_rows, h_rows, w_rows, sem):
        wid = lax.axis_index("s") * 2 + lax.axis_index("c")

        @pl.when(wid == 0)
        def _gather():
            pltpu.sync_copy(t_idx_hbm, t_idx_v)
            pltpu.sync_copy(h_idx_hbm, h_idx_v)
            pltpu.sync_copy(w_idx_hbm, w_idx_v)
            pltpu.async_copy(t_hbm.at[t_idx_v], t_rows, sem).wait()
            pltpu.async_copy(h_hbm.at[h_idx_v], h_rows, sem).wait()
            pltpu.async_copy(w_hbm.at[w_idx_v], w_rows, sem).wait()
            pltpu.sync_copy(t_rows, out.at[pl.ds(0, T_ST)])
            pltpu.sync_copy(h_rows, out.at[pl.ds(H_OFF, H_ST)])
            pltpu.sync_copy(w_rows, out.at[pl.ds(W_OFF, W_ST)])

    return k(t_idx, h_idx, w_idx, temporal, height, width)


def _tc_body(t_ref, h_ref, w_ref, out_ref, scratch, sem):
    i = pl.program_id(0)
    t = i // NB
    hb = i % NB
    buf = jax.lax.rem(i, NBUF)

    dst = out_ref.at[t, pl.ds(hb * BH, BH), :, :]

    @pl.when(i >= NBUF)
    def _wait_prev():
        pltpu.make_async_copy(scratch.at[buf], dst, sem.at[buf]).wait()

    shape = (BH, W_ST, D3)
    t_vec = t_ref[0]
    h_rows = h_ref[:, :]
    w_rows = w_ref[:, :]
    scratch[buf, :, :, 0:D3] = jnp.broadcast_to(t_vec[None, :, :], shape)
    scratch[buf, :, :, D3:2 * D3] = jnp.broadcast_to(h_rows[:, None, :], shape)
    scratch[buf, :, :, 2 * D3:HIDDEN] = jnp.broadcast_to(w_rows[None, :, :], shape)

    pltpu.make_async_copy(scratch.at[buf], dst, sem.at[buf]).start()

    @pl.when(i == GRID - 1)
    def _drain():
        for b in range(NBUF):
            pltpu.make_async_copy(scratch.at[b], dst, sem.at[b]).wait()


def kernel(T, H, W, temporal_embed, height_embed, width_embed):
    t_idx = jnp.arange(T_ST, dtype=jnp.int32) + jnp.asarray(T, jnp.int32) - T_ST
    h_idx = jnp.arange(H_ST, dtype=jnp.int32) + jnp.asarray(H, jnp.int32) - H_ST
    w_idx = jnp.arange(W_ST, dtype=jnp.int32) + jnp.asarray(W, jnp.int32) - W_ST

    emb = _sc_gather(
        t_idx, h_idx, w_idx, temporal_embed, height_embed, width_embed)

    out4 = pl.pallas_call(
        _tc_body,
        grid=(GRID,),
        in_specs=[
            pl.BlockSpec((1, 1, D3), lambda i: (i // NB, 0, 0)),
            pl.BlockSpec((BH, D3), lambda i: (H_OFF // BH + i % NB, 0)),
            pl.BlockSpec((W_ST, D3), lambda i: (W_OFF // W_ST, 0)),
        ],
        out_specs=pl.BlockSpec(memory_space=pl.ANY),
        scratch_shapes=[
            pltpu.VMEM((NBUF, BH, W_ST, HIDDEN), jnp.float32),
            pltpu.SemaphoreType.DMA((NBUF,)),
        ],
        out_shape=jax.ShapeDtypeStruct((T_ST, H_ST, W_ST, HIDDEN), jnp.float32),
    )(emb.reshape(-1, 1, D3), emb, emb)
    return out4.reshape(T_ST * H_ST * W_ST, HIDDEN)


# speedup vs baseline: 1.0186x; 1.0186x over previous
import functools

import jax
import jax.numpy as jnp
from jax import lax
from jax.experimental import pallas as pl
from jax.experimental.pallas import tpu as pltpu
from jax.experimental.pallas import tpu_sc as plsc

T_ST, H_ST, W_ST = 16, 64, 64
HIDDEN = 768
D3 = HIDDEN // 3
DHW = HIDDEN - D3
BH = 16
NB = H_ST // BH
NBUF = 4
GRID = T_ST * NB
NBUF_B = 3


def _sc_gather_t(t_idx, temporal):
    mesh = plsc.VectorSubcoreMesh(core_axis_name="c", subcore_axis_name="s")

    @functools.partial(
        pl.kernel,
        mesh=mesh,
        out_type=jax.ShapeDtypeStruct((T_ST, D3), jnp.float32),
        scratch_types=(
            pltpu.VMEM((T_ST,), jnp.int32),
            pltpu.VMEM((T_ST, D3), jnp.float32),
            pltpu.SemaphoreType.DMA,
        ),
    )
    def k(t_idx_hbm, t_hbm, t_out, t_idx_v, t_rows, sem):
        wid = lax.axis_index("s") * 2 + lax.axis_index("c")

        @pl.when(wid == 0)
        def _gather():
            pltpu.sync_copy(t_idx_hbm, t_idx_v)
            pltpu.async_copy(t_hbm.at[t_idx_v], t_rows, sem).wait()
            pltpu.sync_copy(t_rows, t_out)

    return k(t_idx, temporal)


def _tc_hw_body(offs_ref, h_ref, w_ref, out_ref, scratch, sem):
    del offs_ref
    i = pl.program_id(0)
    t = i // NB
    hb = i % NB
    buf = jax.lax.rem(i, NBUF)

    dst = out_ref.at[t, pl.ds(hb * BH, BH), :, pl.ds(D3, DHW)]

    @pl.when(i >= NBUF)
    def _wait_prev():
        pltpu.make_async_copy(scratch.at[buf], dst, sem.at[buf]).wait()

    shape = (BH, W_ST, D3)
    h_rows = h_ref[:, :]
    w_rows = w_ref[:, :]
    scratch[buf, :, :, 0:D3] = jnp.broadcast_to(h_rows[:, None, :], shape)
    scratch[buf, :, :, D3:DHW] = jnp.broadcast_to(w_rows[None, :, :], shape)

    pltpu.make_async_copy(scratch.at[buf], dst, sem.at[buf]).start()

    @pl.when(i == GRID - 1)
    def _drain():
        for b in range(NBUF):
            pltpu.make_async_copy(scratch.at[b], dst, sem.at[b]).wait()


def _tc_t_body(acc_ref, t_ref, out_ref, scratch, sem):
    del acc_ref
    t = pl.program_id(0)
    buf = jax.lax.rem(t, NBUF_B)

    dst = out_ref.at[t, :, :, pl.ds(0, D3)]

    @pl.when(t >= NBUF_B)
    def _wait_prev():
        pltpu.make_async_copy(scratch.at[buf], dst, sem.at[buf]).wait()

    t_vec = t_ref[0]
    scratch[buf, :, :, :] = jnp.broadcast_to(
        t_vec[None, :, :], (H_ST, W_ST, D3))

    pltpu.make_async_copy(scratch.at[buf], dst, sem.at[buf]).start()

    @pl.when(t == T_ST - 1)
    def _drain():
        for b in range(NBUF_B):
            pltpu.make_async_copy(scratch.at[b], dst, sem.at[b]).wait()


def kernel(T, H, W, temporal_embed, height_embed, width_embed):
    t_idx = jnp.arange(T_ST, dtype=jnp.int32) + jnp.asarray(T, jnp.int32) - T_ST
    offs = jnp.stack([
        jnp.asarray(H, jnp.int32) - H_ST,
        jnp.asarray(W, jnp.int32) - W_ST,
    ])

    t_emb = _sc_gather_t(t_idx, temporal_embed)

    hw = pl.pallas_call(
        _tc_hw_body,
        grid_spec=pltpu.PrefetchScalarGridSpec(
            num_scalar_prefetch=1,
            grid=(GRID,),
            in_specs=[
                pl.BlockSpec((BH, D3),
                             lambda i, offs: ((offs[0] + (i % NB) * BH) // BH, 0)),
                pl.BlockSpec((W_ST, D3),
                             lambda i, offs: (offs[1] // W_ST, 0)),
            ],
            out_specs=pl.BlockSpec(memory_space=pl.ANY),
            scratch_shapes=[
                pltpu.VMEM((NBUF, BH, W_ST, DHW), jnp.float32),
                pltpu.SemaphoreType.DMA((NBUF,)),
            ],
        ),
        out_shape=jax.ShapeDtypeStruct((T_ST, H_ST, W_ST, HIDDEN), jnp.float32),
    )(offs, height_embed, width_embed)

    out4 = pl.pallas_call(
        _tc_t_body,
        grid=(T_ST,),
        in_specs=[
            pl.BlockSpec(memory_space=pl.ANY),
            pl.BlockSpec((1, 1, D3), lambda t: (t, 0, 0)),
        ],
        out_specs=pl.BlockSpec(memory_space=pl.ANY),
        scratch_shapes=[
            pltpu.VMEM((NBUF_B, H_ST, W_ST, D3), jnp.float32),
            pltpu.SemaphoreType.DMA((NBUF_B,)),
        ],
        out_shape=jax.ShapeDtypeStruct((T_ST, H_ST, W_ST, HIDDEN), jnp.float32),
        input_output_aliases={0: 0},
    )(hw, t_emb.reshape(-1, 1, D3))
    return out4.reshape(T_ST * H_ST * W_ST, HIDDEN)


# speedup vs baseline: 1.0375x; 1.0186x over previous
import functools

import jax
import jax.numpy as jnp
from jax import lax
from jax.experimental import pallas as pl
from jax.experimental.pallas import tpu as pltpu
from jax.experimental.pallas import tpu_sc as plsc

T_ST, H_ST, W_ST = 16, 64, 64
HIDDEN = 768
D3 = HIDDEN // 3
BH = 16
NB = H_ST // BH
NBUF = 4
GRID_A = (T_ST - 1) * NB
NBUF_B = 2


def _sc_gather(t_idx, h_idx, w_idx, temporal, height, width):
    mesh = plsc.VectorSubcoreMesh(core_axis_name="c", subcore_axis_name="s")

    @functools.partial(
        pl.kernel,
        mesh=mesh,
        out_type=(
            jax.ShapeDtypeStruct((T_ST, D3), jnp.float32),
            jax.ShapeDtypeStruct((H_ST, D3), jnp.float32),
            jax.ShapeDtypeStruct((W_ST, D3), jnp.float32),
        ),
        scratch_types=(
            pltpu.VMEM((T_ST,), jnp.int32),
            pltpu.VMEM((H_ST,), jnp.int32),
            pltpu.VMEM((W_ST,), jnp.int32),
            pltpu.VMEM((T_ST, D3), jnp.float32),
            pltpu.VMEM((H_ST, D3), jnp.float32),
            pltpu.VMEM((W_ST, D3), jnp.float32),
            pltpu.SemaphoreType.DMA,
        ),
    )
    def k(t_idx_hbm, h_idx_hbm, w_idx_hbm, t_hbm, h_hbm, w_hbm,
          t_out, h_out, w_out,
          t_idx_v, h_idx_v, w_idx_v, t_rows, h_rows, w_rows, sem):
        wid = lax.axis_index("s") * 2 + lax.axis_index("c")

        @pl.when(wid == 0)
        def _gather_t():
            pltpu.sync_copy(t_idx_hbm, t_idx_v)
            pltpu.async_copy(t_hbm.at[t_idx_v], t_rows, sem).wait()
            pltpu.sync_copy(t_rows, t_out)

        @pl.when(wid == 1)
        def _gather_h():
            pltpu.sync_copy(h_idx_hbm, h_idx_v)
            pltpu.async_copy(h_hbm.at[h_idx_v], h_rows, sem).wait()
            pltpu.sync_copy(h_rows, h_out)

        @pl.when(wid == 2)
        def _gather_w():
            pltpu.sync_copy(w_idx_hbm, w_idx_v)
            pltpu.async_copy(w_hbm.at[w_idx_v], w_rows, sem).wait()
            pltpu.sync_copy(w_rows, w_out)

    return k(t_idx, h_idx, w_idx, temporal, height, width)


def _tc_bulk_body(offs_ref, t_ref, h_ref, w_ref, out_ref, scratch, sem):
    del offs_ref
    i = pl.program_id(0)
    t = 1 + i // NB
    hb = i % NB
    buf = jax.lax.rem(i, NBUF)

    dst = out_ref.at[t, pl.ds(hb * BH, BH), :, :]

    @pl.when(i >= NBUF)
    def _wait_prev():
        pltpu.make_async_copy(scratch.at[buf], dst, sem.at[buf]).wait()

    shape = (BH, W_ST, D3)
    t_vec = t_ref[0]
    h_rows = h_ref[:, :]
    w_rows = w_ref[:, :]
    scratch[buf, :, :, 0:D3] = jnp.broadcast_to(t_vec[None, :, :], shape)
    scratch[buf, :, :, D3:2 * D3] = jnp.broadcast_to(h_rows[:, None, :], shape)
    scratch[buf, :, :, 2 * D3:HIDDEN] = jnp.broadcast_to(w_rows[None, :, :], shape)

    pltpu.make_async_copy(scratch.at[buf], dst, sem.at[buf]).start()

    @pl.when(i == GRID_A - 1)
    def _drain():
        for b in range(NBUF):
            pltpu.make_async_copy(scratch.at[b], dst, sem.at[b]).wait()


def _tc_slab0_body(acc_ref, t_ref, h_ref, w_ref, out_ref, scratch, sem):
    del acc_ref
    hb = pl.program_id(0)
    buf = jax.lax.rem(hb, NBUF_B)

    dst = out_ref.at[0, pl.ds(hb * BH, BH), :, :]

    @pl.when(hb >= NBUF_B)
    def _wait_prev():
        pltpu.make_async_copy(scratch.at[buf], dst, sem.at[buf]).wait()

    shape = (BH, W_ST, D3)
    t_vec = t_ref[0]
    h_rows = h_ref[:, :]
    w_rows = w_ref[:, :]
    scratch[buf, :, :, 0:D3] = jnp.broadcast_to(t_vec[None, :, :], shape)
    scratch[buf, :, :, D3:2 * D3] = jnp.broadcast_to(h_rows[:, None, :], shape)
    scratch[buf, :, :, 2 * D3:HIDDEN] = jnp.broadcast_to(w_rows[None, :, :], shape)

    pltpu.make_async_copy(scratch.at[buf], dst, sem.at[buf]).start()

    @pl.when(hb == NB - 1)
    def _drain():
        for b in range(NBUF_B):
            pltpu.make_async_copy(scratch.at[b], dst, sem.at[b]).wait()


def kernel(T, H, W, temporal_embed, height_embed, width_embed):
    t_idx = jnp.arange(T_ST, dtype=jnp.int32) + jnp.asarray(T, jnp.int32) - T_ST
    h_idx = jnp.arange(H_ST, dtype=jnp.int32) + jnp.asarray(H, jnp.int32) - H_ST
    w_idx = jnp.arange(W_ST, dtype=jnp.int32) + jnp.asarray(W, jnp.int32) - W_ST
    offs = jnp.stack([
        jnp.asarray(T, jnp.int32) - T_ST,
        jnp.asarray(H, jnp.int32) - H_ST,
        jnp.asarray(W, jnp.int32) - W_ST,
    ])

    t_emb, h_emb, w_emb = _sc_gather(
        t_idx, h_idx, w_idx, temporal_embed, height_embed, width_embed)

    bulk = pl.pallas_call(
        _tc_bulk_body,
        grid_spec=pltpu.PrefetchScalarGridSpec(
            num_scalar_prefetch=1,
            grid=(GRID_A,),
            in_specs=[
                pl.BlockSpec((1, 1, D3),
                             lambda i, offs: (offs[0] + 1 + i // NB, 0, 0)),
                pl.BlockSpec((BH, D3),
                             lambda i, offs: ((offs[1] + (i % NB) * BH) // BH, 0)),
                pl.BlockSpec((W_ST, D3),
                             lambda i, offs: (offs[2] // W_ST, 0)),
            ],
            out_specs=pl.BlockSpec(memory_space=pl.ANY),
            scratch_shapes=[
                pltpu.VMEM((NBUF, BH, W_ST, HIDDEN), jnp.float32),
                pltpu.SemaphoreType.DMA((NBUF,)),
            ],
        ),
        out_shape=jax.ShapeDtypeStruct((T_ST, H_ST, W_ST, HIDDEN), jnp.float32),
    )(offs, temporal_embed.reshape(-1, 1, D3), height_embed, width_embed)

    out4 = pl.pallas_call(
        _tc_slab0_body,
        grid=(NB,),
        in_specs=[
            pl.BlockSpec(memory_space=pl.ANY),
            pl.BlockSpec((1, 1, D3), lambda hb: (0, 0, 0)),
            pl.BlockSpec((BH, D3), lambda hb: (hb, 0)),
            pl.BlockSpec((W_ST, D3), lambda hb: (0, 0)),
        ],
        out_specs=pl.BlockSpec(memory_space=pl.ANY),
        scratch_shapes=[
            pltpu.VMEM((NBUF_B, BH, W_ST, HIDDEN), jnp.float32),
            pltpu.SemaphoreType.DMA((NBUF_B,)),
        ],
        out_shape=jax.ShapeDtypeStruct((T_ST, H_ST, W_ST, HIDDEN), jnp.float32),
        input_output_aliases={0: 0},
    )(bulk, t_emb.reshape(-1, 1, D3), h_emb, w_emb)
    return out4.reshape(T_ST * H_ST * W_ST, HIDDEN)


# speedup vs baseline: 1.0476x; 1.0097x over previous
import functools

import jax
import jax.numpy as jnp
from jax import lax
from jax.experimental import pallas as pl
from jax.experimental.pallas import tpu as pltpu
from jax.experimental.pallas import tpu_sc as plsc

T_ST, H_ST, W_ST = 16, 64, 64
HIDDEN = 768
D3 = HIDDEN // 3
BH = 16
NB = H_ST // BH
NBUF = 6
GRID = T_ST * NB


def _sc_gather(t_idx, h_idx, w_idx, temporal, height, width):
    mesh = plsc.VectorSubcoreMesh(core_axis_name="c", subcore_axis_name="s")

    @functools.partial(
        pl.kernel,
        mesh=mesh,
        out_type=(
            jax.ShapeDtypeStruct((T_ST, D3), jnp.float32),
            jax.ShapeDtypeStruct((H_ST, D3), jnp.float32),
            jax.ShapeDtypeStruct((W_ST, D3), jnp.float32),
        ),
        scratch_types=(
            pltpu.VMEM((T_ST,), jnp.int32),
            pltpu.VMEM((H_ST,), jnp.int32),
            pltpu.VMEM((W_ST,), jnp.int32),
            pltpu.VMEM((T_ST, D3), jnp.float32),
            pltpu.VMEM((H_ST, D3), jnp.float32),
            pltpu.VMEM((W_ST, D3), jnp.float32),
            pltpu.SemaphoreType.DMA,
        ),
    )
    def k(t_idx_hbm, h_idx_hbm, w_idx_hbm, t_hbm, h_hbm, w_hbm,
          t_out, h_out, w_out,
          t_idx_v, h_idx_v, w_idx_v, t_rows, h_rows, w_rows, sem):
        wid = lax.axis_index("s") * 2 + lax.axis_index("c")

        @pl.when(wid == 0)
        def _gather_t():
            pltpu.sync_copy(t_idx_hbm, t_idx_v)
            pltpu.async_copy(t_hbm.at[t_idx_v], t_rows, sem).wait()
            pltpu.sync_copy(t_rows, t_out)

        @pl.when(wid == 1)
        def _gather_h():
            pltpu.sync_copy(h_idx_hbm, h_idx_v)
            pltpu.async_copy(h_hbm.at[h_idx_v], h_rows, sem).wait()
            pltpu.sync_copy(h_rows, h_out)

        @pl.when(wid == 2)
        def _gather_w():
            pltpu.sync_copy(w_idx_hbm, w_idx_v)
            pltpu.async_copy(w_hbm.at[w_idx_v], w_rows, sem).wait()
            pltpu.sync_copy(w_rows, w_out)

    return k(t_idx, h_idx, w_idx, temporal, height, width)


def _tc_body(t_ref, h_ref, w_ref, out_ref, scratch, sem):
    i = pl.program_id(0)
    t = i // NB
    hb = i % NB
    buf = jax.lax.rem(i, NBUF)

    dst = out_ref.at[t, pl.ds(hb * BH, BH), :, :]

    @pl.when(i >= NBUF)
    def _wait_prev():
        pltpu.make_async_copy(scratch.at[buf], dst, sem.at[buf]).wait()

    shape = (BH, W_ST, D3)
    t_vec = t_ref[0]
    h_rows = h_ref[:, :]
    w_rows = w_ref[:, :]
    scratch[buf, :, :, 0:D3] = jnp.broadcast_to(t_vec[None, :, :], shape)
    scratch[buf, :, :, D3:2 * D3] = jnp.broadcast_to(h_rows[:, None, :], shape)
    scratch[buf, :, :, 2 * D3:HIDDEN] = jnp.broadcast_to(w_rows[None, :, :], shape)

    pltpu.make_async_copy(scratch.at[buf], dst, sem.at[buf]).start()

    @pl.when(i == GRID - 1)
    def _drain():
        for b in range(NBUF):
            pltpu.make_async_copy(scratch.at[b], dst, sem.at[b]).wait()


def kernel(T, H, W, temporal_embed, height_embed, width_embed):
    t_idx = jnp.arange(T_ST, dtype=jnp.int32) + jnp.asarray(T, jnp.int32) - T_ST
    h_idx = jnp.arange(H_ST, dtype=jnp.int32) + jnp.asarray(H, jnp.int32) - H_ST
    w_idx = jnp.arange(W_ST, dtype=jnp.int32) + jnp.asarray(W, jnp.int32) - W_ST

    t_emb, h_emb, w_emb = _sc_gather(
        t_idx, h_idx, w_idx, temporal_embed, height_embed, width_embed)

    out4 = pl.pallas_call(
        _tc_body,
        grid=(GRID,),
        in_specs=[
            pl.BlockSpec((1, 1, D3), lambda i: (i // NB, 0, 0)),
            pl.BlockSpec((BH, D3), lambda i: (i % NB, 0)),
            pl.BlockSpec((W_ST, D3), lambda i: (0, 0)),
        ],
        out_specs=pl.BlockSpec(memory_space=pl.ANY),
        scratch_shapes=[
            pltpu.VMEM((NBUF, BH, W_ST, HIDDEN), jnp.float32),
            pltpu.SemaphoreType.DMA((NBUF,)),
        ],
        out_shape=jax.ShapeDtypeStruct((T_ST, H_ST, W_ST, HIDDEN), jnp.float32),
    )(t_emb.reshape(-1, 1, D3), h_emb, w_emb)
    return out4.reshape(T_ST * H_ST * W_ST, HIDDEN)


# speedup vs baseline: 1.0632x; 1.0149x over previous
import functools

import jax
import jax.numpy as jnp
from jax import lax
from jax.experimental import pallas as pl
from jax.experimental.pallas import tpu as pltpu
from jax.experimental.pallas import tpu_sc as plsc

T_ST, H_ST, W_ST = 16, 64, 64
HIDDEN = 768
D3 = HIDDEN // 3
BH = 16
NB = H_ST // BH
NBUF = 6
GRID = T_ST * NB


def _sc_gather(t_idx, h_idx, w_idx, temporal, height, width):
    mesh = plsc.VectorSubcoreMesh(core_axis_name="c", subcore_axis_name="s", num_cores=1)

    @functools.partial(
        pl.kernel,
        mesh=mesh,
        out_type=(
            jax.ShapeDtypeStruct((T_ST, D3), jnp.float32),
            jax.ShapeDtypeStruct((H_ST, D3), jnp.float32),
            jax.ShapeDtypeStruct((W_ST, D3), jnp.float32),
        ),
        scratch_types=(
            pltpu.VMEM((T_ST,), jnp.int32),
            pltpu.VMEM((H_ST,), jnp.int32),
            pltpu.VMEM((W_ST,), jnp.int32),
            pltpu.VMEM((T_ST, D3), jnp.float32),
            pltpu.VMEM((H_ST, D3), jnp.float32),
            pltpu.VMEM((W_ST, D3), jnp.float32),
            pltpu.SemaphoreType.DMA,
        ),
    )
    def k(t_idx_hbm, h_idx_hbm, w_idx_hbm, t_hbm, h_hbm, w_hbm,
          t_out, h_out, w_out,
          t_idx_v, h_idx_v, w_idx_v, t_rows, h_rows, w_rows, sem):
        wid = lax.axis_index("s")

        @pl.when(wid == 0)
        def _gather_t():
            pltpu.sync_copy(t_idx_hbm, t_idx_v)
            pltpu.async_copy(t_hbm.at[t_idx_v], t_rows, sem).wait()
            pltpu.sync_copy(t_rows, t_out)

        @pl.when(wid == 1)
        def _gather_h():
            pltpu.sync_copy(h_idx_hbm, h_idx_v)
            pltpu.async_copy(h_hbm.at[h_idx_v], h_rows, sem).wait()
            pltpu.sync_copy(h_rows, h_out)

        @pl.when(wid == 2)
        def _gather_w():
            pltpu.sync_copy(w_idx_hbm, w_idx_v)
            pltpu.async_copy(w_hbm.at[w_idx_v], w_rows, sem).wait()
            pltpu.sync_copy(w_rows, w_out)

    return k(t_idx, h_idx, w_idx, temporal, height, width)


def _tc_body(t_ref, h_ref, w_ref, out_ref, scratch, sem):
    i = pl.program_id(0)
    t = i // NB
    hb = i % NB
    buf = jax.lax.rem(i, NBUF)

    dst = out_ref.at[t, pl.ds(hb * BH, BH), :, :]

    @pl.when(i >= NBUF)
    def _wait_prev():
        pltpu.make_async_copy(scratch.at[buf], dst, sem.at[buf]).wait()

    shape = (BH, W_ST, D3)
    t_vec = t_ref[0]
    h_rows = h_ref[:, :]
    w_rows = w_ref[:, :]
    scratch[buf, :, :, 0:D3] = jnp.broadcast_to(t_vec[None, :, :], shape)
    scratch[buf, :, :, D3:2 * D3] = jnp.broadcast_to(h_rows[:, None, :], shape)
    scratch[buf, :, :, 2 * D3:HIDDEN] = jnp.broadcast_to(w_rows[None, :, :], shape)

    pltpu.make_async_copy(scratch.at[buf], dst, sem.at[buf]).start()

    @pl.when(i == GRID - 1)
    def _drain():
        for b in range(NBUF):
            pltpu.make_async_copy(scratch.at[b], dst, sem.at[b]).wait()


def kernel(T, H, W, temporal_embed, height_embed, width_embed):
    t_idx = jnp.arange(T_ST, dtype=jnp.int32) + jnp.asarray(T, jnp.int32) - T_ST
    h_idx = jnp.arange(H_ST, dtype=jnp.int32) + jnp.asarray(H, jnp.int32) - H_ST
    w_idx = jnp.arange(W_ST, dtype=jnp.int32) + jnp.asarray(W, jnp.int32) - W_ST

    t_emb, h_emb, w_emb = _sc_gather(
        t_idx, h_idx, w_idx, temporal_embed, height_embed, width_embed)

    out4 = pl.pallas_call(
        _tc_body,
        grid=(GRID,),
        in_specs=[
            pl.BlockSpec((1, 1, D3), lambda i: (i // NB, 0, 0)),
            pl.BlockSpec((BH, D3), lambda i: (i % NB, 0)),
            pl.BlockSpec((W_ST, D3), lambda i: (0, 0)),
        ],
        out_specs=pl.BlockSpec(memory_space=pl.ANY),
        scratch_shapes=[
            pltpu.VMEM((NBUF, BH, W_ST, HIDDEN), jnp.float32),
            pltpu.SemaphoreType.DMA((NBUF,)),
        ],
        out_shape=jax.ShapeDtypeStruct((T_ST, H_ST, W_ST, HIDDEN), jnp.float32),
    )(t_emb.reshape(-1, 1, D3), h_emb, w_emb)
    return out4.reshape(T_ST * H_ST * W_ST, HIDDEN)
